# Initial kernel scaffold; baseline (speedup 1.0000x reference)
import jax
import jax.numpy as jnp
from jax.experimental import pallas as pl


def kernel(x, edge_index, batch, W_proj, b_proj, bn1_g, bn1_b, ggc_w, gru_w_ih, gru_w_hh, gru_b_ih, gru_b_hh, bn2_g, bn2_b, W1, b1, bn3_g, bn3_b, W2, b2, bn4_g, bn4_b, W3, b3):
    raise NotImplementedError("write your pallas kernel here")


# speedup vs baseline: 3.4808x; 3.4808x over previous
import functools

import jax
import jax.numpy as jnp
from jax import lax
from jax.experimental import pallas as pl
from jax.experimental.pallas import tpu as pltpu
from jax.experimental.pallas import tpu_sc as plsc

N = 10000
E = 320000
D = 128
H = 128
H3 = 3 * H
OUT = 2
G = 64
STEPS = 3
EPS = 1e-5

BN = 512
N_PAD = 10240
NBLK = N_PAD // BN

NC = 2
NS = 16
NW = NC * NS
CHUNK = 128
CPW = -(-E // (NW * CHUNK))
E_PAD = NW * CPW * CHUNK
STRIPE = N_PAD // NS


def _dot(a, b):
    return jnp.dot(a, b, preferred_element_type=jnp.float32)


def _sc_edge_segsum(m_pad, src_p, dst_p, zeros_stripe):
    mesh = plsc.VectorSubcoreMesh(core_axis_name="c", subcore_axis_name="s")

    @functools.partial(
        pl.kernel,
        mesh=mesh,
        out_type=jax.ShapeDtypeStruct((NC, N_PAD, H), jnp.float32),
        scratch_types=[
            pltpu.VMEM((CHUNK,), jnp.int32),
            pltpu.VMEM((CHUNK,), jnp.int32),
            pltpu.VMEM((CHUNK, H), jnp.float32),
            pltpu.VMEM_SHARED((N_PAD, H), jnp.float32),
            pltpu.SemaphoreType.DMA,
        ],
    )
    def k(m_hbm, src_hbm, dst_hbm, z_hbm, out_hbm, src_v, dst_v, rows_v,
          acc, sem):
        cid = lax.axis_index("c")
        sid = lax.axis_index("s")
        base_r = sid * STRIPE
        pltpu.sync_copy(z_hbm, acc.at[pl.ds(base_r, STRIPE)])
        plsc.subcore_barrier()

        wid = sid * NC + cid
        ebase = wid * (CPW * CHUNK)

        def body(ci, _):
            off = ebase + ci * CHUNK
            pltpu.sync_copy(src_hbm.at[pl.ds(off, CHUNK)], src_v)
            pltpu.sync_copy(dst_hbm.at[pl.ds(off, CHUNK)], dst_v)
            pltpu.async_copy(m_hbm.at[src_v], rows_v, sem).wait()
            pltpu.sync_copy(rows_v, acc.at[dst_v], add=True)
            return 0

        lax.fori_loop(0, CPW, body, 0)
        plsc.subcore_barrier()
        pltpu.sync_copy(acc.at[pl.ds(base_r, STRIPE)],
                        out_hbm.at[cid, pl.ds(base_r, STRIPE)])

    return k(m_pad, src_p, dst_p, zeros_stripe)


def _tc_proj(x_p, W_proj, b_proj, s1g, s1b, ggc0, whhT, bhh):
    def body(x_ref, wp_ref, bp_ref, g_ref, b_ref, ggc_ref, whh_ref, bhh_ref,
             h_ref, m_ref, gh_ref):
        h = _dot(x_ref[...], wp_ref[...]) + bp_ref[...]
        h = jnp.maximum(h * g_ref[...] + b_ref[...], 0.0)
        h_ref[...] = h
        m_ref[...] = _dot(h, ggc_ref[...])
        gh_ref[...] = _dot(h, whh_ref[...]) + bhh_ref[...]

    full = lambda shape: pl.BlockSpec(shape, lambda i: (0, 0))
    return pl.pallas_call(
        body,
        grid=(NBLK,),
        in_specs=[
            pl.BlockSpec((BN, D), lambda i: (i, 0)),
            full((D, H)), full((1, H)), full((1, H)), full((1, H)),
            full((H, H)), full((H, H3)), full((1, H3)),
        ],
        out_specs=[
            pl.BlockSpec((BN, H), lambda i: (i, 0)),
            pl.BlockSpec((BN, H), lambda i: (i, 0)),
            pl.BlockSpec((BN, H3), lambda i: (i, 0)),
        ],
        out_shape=[
            jax.ShapeDtypeStruct((N_PAD, H), jnp.float32),
            jax.ShapeDtypeStruct((N_PAD, H), jnp.float32),
            jax.ShapeDtypeStruct((N_PAD, H3), jnp.float32),
        ],
    )(x_p, W_proj, b_proj, s1g, s1b, ggc0, whhT, bhh)


def _gru_math(parts, gh, h, wih, bih):
    agg = parts[0] + parts[1]
    gi = _dot(agg, wih) + bih
    r = jax.nn.sigmoid(gi[:, :H] + gh[:, :H])
    z = jax.nn.sigmoid(gi[:, H:2 * H] + gh[:, H:2 * H])
    n = jnp.tanh(gi[:, 2 * H:] + r * gh[:, 2 * H:])
    return (1.0 - z) * n + z * h


def _tc_gru_mid(parts, gh, h, wihT, bih, ggc_next, whhT, bhh):
    def body(p_ref, gh_ref, h_ref, wih_ref, bih_ref, ggc_ref, whh_ref,
             bhh_ref, h_out, m_out, gh_out):
        h_new = _gru_math(p_ref[...], gh_ref[...], h_ref[...], wih_ref[...],
                          bih_ref[...])
        h_out[...] = h_new
        m_out[...] = _dot(h_new, ggc_ref[...])
        gh_out[...] = _dot(h_new, whh_ref[...]) + bhh_ref[...]

    full = lambda shape: pl.BlockSpec(shape, lambda i: (0, 0))
    return pl.pallas_call(
        body,
        grid=(NBLK,),
        in_specs=[
            pl.BlockSpec((NC, BN, H), lambda i: (0, i, 0)),
            pl.BlockSpec((BN, H3), lambda i: (i, 0)),
            pl.BlockSpec((BN, H), lambda i: (i, 0)),
            full((H, H3)), full((1, H3)),
            full((H, H)), full((H, H3)), full((1, H3)),
        ],
        out_specs=[
            pl.BlockSpec((BN, H), lambda i: (i, 0)),
            pl.BlockSpec((BN, H), lambda i: (i, 0)),
            pl.BlockSpec((BN, H3), lambda i: (i, 0)),
        ],
        out_shape=[
            jax.ShapeDtypeStruct((N_PAD, H), jnp.float32),
            jax.ShapeDtypeStruct((N_PAD, H), jnp.float32),
            jax.ShapeDtypeStruct((N_PAD, H3), jnp.float32),
        ],
    )(parts, gh, h, wihT, bih, ggc_next, whhT, bhh)


def _tc_gru_last(parts, gh, h, wihT, bih, s2g, s2b, ind):
    def body(p_ref, gh_ref, h_ref, wih_ref, bih_ref, g_ref, b_ref, ind_ref,
             sums_out, cnt_out):
        h_new = _gru_math(p_ref[...], gh_ref[...], h_ref[...], wih_ref[...],
                          bih_ref[...])
        pfeat = jnp.maximum(h_new * g_ref[...] + b_ref[...], 0.0)
        I = ind_ref[...]

        @pl.when(pl.program_id(0) == 0)
        def _():
            sums_out[...] = jnp.zeros_like(sums_out)
            cnt_out[...] = jnp.zeros_like(cnt_out)

        sums_out[...] += _dot(I, pfeat)
        cnt_out[...] += jnp.broadcast_to(
            jnp.sum(I, axis=1, keepdims=True), (G, H))

    full = lambda shape: pl.BlockSpec(shape, lambda i: (0, 0))
    return pl.pallas_call(
        body,
        grid=(NBLK,),
        in_specs=[
            pl.BlockSpec((NC, BN, H), lambda i: (0, i, 0)),
            pl.BlockSpec((BN, H3), lambda i: (i, 0)),
            pl.BlockSpec((BN, H), lambda i: (i, 0)),
            full((H, H3)), full((1, H3)), full((1, H)), full((1, H)),
            pl.BlockSpec((G, BN), lambda i: (0, i)),
        ],
        out_specs=[
            pl.BlockSpec((G, H), lambda i: (0, 0)),
            pl.BlockSpec((G, H), lambda i: (0, 0)),
        ],
        out_shape=[
            jax.ShapeDtypeStruct((G, H), jnp.float32),
            jax.ShapeDtypeStruct((G, H), jnp.float32),
        ],
    )(parts, gh, h, wihT, bih, s2g, s2b, ind)


def _tc_head(sums, cnt, W1, b1, s3g, s3b, W2p, b2p, s4gp, s4bp, W3p, b3p):
    def body(s_ref, c_ref, w1_ref, b1_ref, g3_ref, b3_ref, w2_ref, b2_ref,
             g4_ref, b4_ref, w3_ref, bo_ref, out_ref):
        pooled = s_ref[...] * (1.0 / jnp.maximum(c_ref[...], 1.0))
        h1 = _dot(pooled, w1_ref[...]) + b1_ref[...]
        h1 = jnp.maximum(h1 * g3_ref[...] + b3_ref[...], 0.0)
        h2 = _dot(h1, w2_ref[...]) + b2_ref[...]
        h2 = jnp.maximum(h2 * g4_ref[...] + b4_ref[...], 0.0)
        out_ref[...] = _dot(h2, w3_ref[...]) + bo_ref[...]

    return pl.pallas_call(
        body,
        out_shape=jax.ShapeDtypeStruct((G, H), jnp.float32),
    )(sums, cnt, W1, b1, s3g, s3b, W2p, b2p, s4gp, s4bp, W3p, b3p)


def kernel(x, edge_index, batch, W_proj, b_proj, bn1_g, bn1_b, ggc_w,
           gru_w_ih, gru_w_hh, gru_b_ih, gru_b_hh, bn2_g, bn2_b, W1, b1,
           bn3_g, bn3_b, W2, b2, bn4_g, bn4_b, W3, b3):
    f32 = jnp.float32
    inv = 1.0 / jnp.sqrt(jnp.asarray(1.0 + EPS, f32))
    row = lambda v: v.reshape(1, -1).astype(f32)

    s1g, s1b = row(bn1_g * inv), row(bn1_b)
    s2g, s2b = row(bn2_g * inv), row(bn2_b)
    s3g, s3b = row(bn3_g * inv), row(bn3_b)

    wihT = gru_w_ih.T
    whhT = gru_w_hh.T
    bih, bhh = row(gru_b_ih), row(gru_b_hh)

    x_p = jnp.pad(x, ((0, N_PAD - N), (0, 0)))
    batch_p = jnp.concatenate(
        [batch, jnp.full((N_PAD - N,), G, jnp.int32)])
    ind = (jnp.arange(G, dtype=jnp.int32)[:, None]
           == batch_p[None, :]).astype(f32)

    src_p = jnp.concatenate(
        [edge_index[0], jnp.zeros((E_PAD - E,), jnp.int32)])
    dst_p = jnp.concatenate(
        [edge_index[1], jnp.full((E_PAD - E,), N, jnp.int32)])
    zeros_stripe = jnp.zeros((STRIPE, H), f32)

    W2p = jnp.pad(W2, ((0, 0), (0, H - H // 2)))
    b2p = jnp.pad(row(b2), ((0, 0), (0, H - H // 2)))
    s4gp = jnp.pad(row(bn4_g * inv), ((0, 0), (0, H - H // 2)))
    s4bp = jnp.pad(row(bn4_b), ((0, 0), (0, H - H // 2)))
    W3p = jnp.pad(W3, ((0, H - H // 2), (0, H - OUT)))
    b3p = jnp.pad(row(b3), ((0, 0), (0, H - OUT)))

    h, m, gh = _tc_proj(x_p, W_proj, row(b_proj), s1g, s1b, ggc_w[0],
                        whhT, bhh)
    for i in range(STEPS):
        parts = _sc_edge_segsum(m, src_p, dst_p, zeros_stripe)
        if i < STEPS - 1:
            h, m, gh = _tc_gru_mid(parts, gh, h, wihT, bih, ggc_w[i + 1],
                                   whhT, bhh)
        else:
            sums, cnt = _tc_gru_last(parts, gh, h, wihT, bih, s2g, s2b, ind)

    out = _tc_head(sums, cnt, W1, row(b1), s3g, s3b, W2p, b2p, s4gp, s4bp,
                   W3p, b3p)
    return out[:, :OUT]


# speedup vs baseline: 3.6881x; 1.0596x over previous
import functools

import jax
import jax.numpy as jnp
from jax import lax
from jax.experimental import pallas as pl
from jax.experimental.pallas import tpu as pltpu
from jax.experimental.pallas import tpu_sc as plsc

N = 10000
E = 320000
D = 128
H = 128
H3 = 3 * H
OUT = 2
G = 64
STEPS = 3
EPS = 1e-5

BN = 512
N_PAD = 10240
NBLK = N_PAD // BN

NC = 2
NS = 16
NW = NC * NS
CHUNK = 128
NBUF = 2
CPW = 2 * NBUF * (-(-E // (NW * CHUNK * 2 * NBUF)))
HCPW = CPW // 2
E_PAD = NW * CPW * CHUNK
STRIPE = N_PAD // NS


def _dot(a, b):
    return jnp.dot(a, b, preferred_element_type=jnp.float32)


def _sc_edge_segsum(m_pad, src3, dst3, zeros_stripe):
    mesh = plsc.VectorSubcoreMesh(core_axis_name="c", subcore_axis_name="s")

    @functools.partial(
        pl.kernel,
        mesh=mesh,
        out_type=jax.ShapeDtypeStruct((NC, N_PAD, H), jnp.float32),
        scratch_types=[
            pltpu.VMEM((HCPW, CHUNK), jnp.int32),
            pltpu.VMEM((HCPW, CHUNK), jnp.int32),
            pltpu.VMEM((NBUF, CHUNK, H), jnp.float32),
            pltpu.VMEM_SHARED((N_PAD, H), jnp.float32),
            pltpu.SemaphoreType.DMA,
            pltpu.SemaphoreType.DMA,
        ],
    )
    def k(m_hbm, src_hbm, dst_hbm, z_hbm, out_hbm, src_v, dst_v, rows_v,
          acc, *sems):
        cid = lax.axis_index("c")
        sid = lax.axis_index("s")
        base_r = sid * STRIPE
        wid = sid * NC + cid
        pltpu.sync_copy(z_hbm, acc.at[pl.ds(base_r, STRIPE)])
        plsc.subcore_barrier()

        for half in range(2):
            pltpu.sync_copy(src_hbm.at[wid, pl.ds(half * HCPW, HCPW)],
                            src_v)
            pltpu.sync_copy(dst_hbm.at[wid, pl.ds(half * HCPW, HCPW)],
                            dst_v)
            for b in range(NBUF):
                pltpu.async_copy(m_hbm.at[src_v.at[b]], rows_v.at[b],
                                 sems[b])

            def outer(g, _):
                for b in range(NBUF):
                    c = g * NBUF + b
                    pltpu.make_async_copy(m_hbm.at[pl.ds(0, CHUNK)],
                                          rows_v.at[b], sems[b]).wait()
                    pltpu.sync_copy(rows_v.at[b], acc.at[dst_v.at[c]],
                                    add=True)

                    @pl.when(c + NBUF < HCPW)
                    def _():
                        pltpu.async_copy(m_hbm.at[src_v.at[c + NBUF]],
                                         rows_v.at[b], sems[b])
                return 0

            lax.fori_loop(0, HCPW // NBUF, outer, 0)
        plsc.subcore_barrier()
        pltpu.sync_copy(acc.at[pl.ds(base_r, STRIPE)],
                        out_hbm.at[cid, pl.ds(base_r, STRIPE)])

    return k(m_pad, src3, dst3, zeros_stripe)


def _tc_proj(x_p, W_proj, b_proj, s1g, s1b, ggc0, whhT, bhh):
    def body(x_ref, wp_ref, bp_ref, g_ref, b_ref, ggc_ref, whh_ref, bhh_ref,
             h_ref, m_ref, gh_ref):
        h = _dot(x_ref[...], wp_ref[...]) + bp_ref[...]
        h = jnp.maximum(h * g_ref[...] + b_ref[...], 0.0)
        h_ref[...] = h
        m_ref[...] = _dot(h, ggc_ref[...])
        gh_ref[...] = _dot(h, whh_ref[...]) + bhh_ref[...]

    full = lambda shape: pl.BlockSpec(shape, lambda i: (0, 0))
    return pl.pallas_call(
        body,
        grid=(NBLK,),
        in_specs=[
            pl.BlockSpec((BN, D), lambda i: (i, 0)),
            full((D, H)), full((1, H)), full((1, H)), full((1, H)),
            full((H, H)), full((H, H3)), full((1, H3)),
        ],
        out_specs=[
            pl.BlockSpec((BN, H), lambda i: (i, 0)),
            pl.BlockSpec((BN, H), lambda i: (i, 0)),
            pl.BlockSpec((BN, H3), lambda i: (i, 0)),
        ],
        out_shape=[
            jax.ShapeDtypeStruct((N_PAD, H), jnp.float32),
            jax.ShapeDtypeStruct((N_PAD, H), jnp.float32),
            jax.ShapeDtypeStruct((N_PAD, H3), jnp.float32),
        ],
    )(x_p, W_proj, b_proj, s1g, s1b, ggc0, whhT, bhh)


def _gru_math(parts, gh, h, wih, bih):
    agg = parts[0] + parts[1]
    gi = _dot(agg, wih) + bih
    r = jax.nn.sigmoid(gi[:, :H] + gh[:, :H])
    z = jax.nn.sigmoid(gi[:, H:2 * H] + gh[:, H:2 * H])
    n = jnp.tanh(gi[:, 2 * H:] + r * gh[:, 2 * H:])
    return (1.0 - z) * n + z * h


def _tc_gru_mid(parts, gh, h, wihT, bih, ggc_next, whhT, bhh):
    def body(p_ref, gh_ref, h_ref, wih_ref, bih_ref, ggc_ref, whh_ref,
             bhh_ref, h_out, m_out, gh_out):
        h_new = _gru_math(p_ref[...], gh_ref[...], h_ref[...], wih_ref[...],
                          bih_ref[...])
        h_out[...] = h_new
        m_out[...] = _dot(h_new, ggc_ref[...])
        gh_out[...] = _dot(h_new, whh_ref[...]) + bhh_ref[...]

    full = lambda shape: pl.BlockSpec(shape, lambda i: (0, 0))
    return pl.pallas_call(
        body,
        grid=(NBLK,),
        in_specs=[
            pl.BlockSpec((NC, BN, H), lambda i: (0, i, 0)),
            pl.BlockSpec((BN, H3), lambda i: (i, 0)),
            pl.BlockSpec((BN, H), lambda i: (i, 0)),
            full((H, H3)), full((1, H3)),
            full((H, H)), full((H, H3)), full((1, H3)),
        ],
        out_specs=[
            pl.BlockSpec((BN, H), lambda i: (i, 0)),
            pl.BlockSpec((BN, H), lambda i: (i, 0)),
            pl.BlockSpec((BN, H3), lambda i: (i, 0)),
        ],
        out_shape=[
            jax.ShapeDtypeStruct((N_PAD, H), jnp.float32),
            jax.ShapeDtypeStruct((N_PAD, H), jnp.float32),
            jax.ShapeDtypeStruct((N_PAD, H3), jnp.float32),
        ],
    )(parts, gh, h, wihT, bih, ggc_next, whhT, bhh)


def _tc_gru_last(parts, gh, h, wihT, bih, s2g, s2b, ind):
    def body(p_ref, gh_ref, h_ref, wih_ref, bih_ref, g_ref, b_ref, ind_ref,
             sums_out, cnt_out):
        h_new = _gru_math(p_ref[...], gh_ref[...], h_ref[...], wih_ref[...],
                          bih_ref[...])
        pfeat = jnp.maximum(h_new * g_ref[...] + b_ref[...], 0.0)
        I = ind_ref[...]

        @pl.when(pl.program_id(0) == 0)
        def _():
            sums_out[...] = jnp.zeros_like(sums_out)
            cnt_out[...] = jnp.zeros_like(cnt_out)

        sums_out[...] += _dot(I, pfeat)
        cnt_out[...] += jnp.broadcast_to(
            jnp.sum(I, axis=1, keepdims=True), (G, H))

    full = lambda shape: pl.BlockSpec(shape, lambda i: (0, 0))
    return pl.pallas_call(
        body,
        grid=(NBLK,),
        in_specs=[
            pl.BlockSpec((NC, BN, H), lambda i: (0, i, 0)),
            pl.BlockSpec((BN, H3), lambda i: (i, 0)),
            pl.BlockSpec((BN, H), lambda i: (i, 0)),
            full((H, H3)), full((1, H3)), full((1, H)), full((1, H)),
            pl.BlockSpec((G, BN), lambda i: (0, i)),
        ],
        out_specs=[
            pl.BlockSpec((G, H), lambda i: (0, 0)),
            pl.BlockSpec((G, H), lambda i: (0, 0)),
        ],
        out_shape=[
            jax.ShapeDtypeStruct((G, H), jnp.float32),
            jax.ShapeDtypeStruct((G, H), jnp.float32),
        ],
    )(parts, gh, h, wihT, bih, s2g, s2b, ind)


def _tc_head(sums, cnt, W1, b1, s3g, s3b, W2p, b2p, s4gp, s4bp, W3p, b3p):
    def body(s_ref, c_ref, w1_ref, b1_ref, g3_ref, b3_ref, w2_ref, b2_ref,
             g4_ref, b4_ref, w3_ref, bo_ref, out_ref):
        pooled = s_ref[...] * (1.0 / jnp.maximum(c_ref[...], 1.0))
        h1 = _dot(pooled, w1_ref[...]) + b1_ref[...]
        h1 = jnp.maximum(h1 * g3_ref[...] + b3_ref[...], 0.0)
        h2 = _dot(h1, w2_ref[...]) + b2_ref[...]
        h2 = jnp.maximum(h2 * g4_ref[...] + b4_ref[...], 0.0)
        out_ref[...] = _dot(h2, w3_ref[...]) + bo_ref[...]

    return pl.pallas_call(
        body,
        out_shape=jax.ShapeDtypeStruct((G, H), jnp.float32),
    )(sums, cnt, W1, b1, s3g, s3b, W2p, b2p, s4gp, s4bp, W3p, b3p)


def kernel(x, edge_index, batch, W_proj, b_proj, bn1_g, bn1_b, ggc_w,
           gru_w_ih, gru_w_hh, gru_b_ih, gru_b_hh, bn2_g, bn2_b, W1, b1,
           bn3_g, bn3_b, W2, b2, bn4_g, bn4_b, W3, b3):
    f32 = jnp.float32
    inv = 1.0 / jnp.sqrt(jnp.asarray(1.0 + EPS, f32))
    row = lambda v: v.reshape(1, -1).astype(f32)

    s1g, s1b = row(bn1_g * inv), row(bn1_b)
    s2g, s2b = row(bn2_g * inv), row(bn2_b)
    s3g, s3b = row(bn3_g * inv), row(bn3_b)

    wihT = gru_w_ih.T
    whhT = gru_w_hh.T
    bih, bhh = row(gru_b_ih), row(gru_b_hh)

    x_p = jnp.pad(x, ((0, N_PAD - N), (0, 0)))
    batch_p = jnp.concatenate(
        [batch, jnp.full((N_PAD - N,), G, jnp.int32)])
    ind = (jnp.arange(G, dtype=jnp.int32)[:, None]
           == batch_p[None, :]).astype(f32)

    src3 = jnp.concatenate(
        [edge_index[0], jnp.zeros((E_PAD - E,), jnp.int32)]).reshape(
            NW, CPW, CHUNK)
    dst3 = jnp.concatenate(
        [edge_index[1], jnp.full((E_PAD - E,), N, jnp.int32)]).reshape(
            NW, CPW, CHUNK)
    zeros_stripe = jnp.zeros((STRIPE, H), f32)

    W2p = jnp.pad(W2, ((0, 0), (0, H - H // 2)))
    b2p = jnp.pad(row(b2), ((0, 0), (0, H - H // 2)))
    s4gp = jnp.pad(row(bn4_g * inv), ((0, 0), (0, H - H // 2)))
    s4bp = jnp.pad(row(bn4_b), ((0, 0), (0, H - H // 2)))
    W3p = jnp.pad(W3, ((0, H - H // 2), (0, H - OUT)))
    b3p = jnp.pad(row(b3), ((0, 0), (0, H - OUT)))

    h, m, gh = _tc_proj(x_p, W_proj, row(b_proj), s1g, s1b, ggc_w[0],
                        whhT, bhh)
    for i in range(STEPS):
        parts = _sc_edge_segsum(m, src3, dst3, zeros_stripe)
        if i < STEPS - 1:
            h, m, gh = _tc_gru_mid(parts, gh, h, wihT, bih, ggc_w[i + 1],
                                   whhT, bhh)
        else:
            sums, cnt = _tc_gru_last(parts, gh, h, wihT, bih, s2g, s2b, ind)

    out = _tc_head(sums, cnt, W1, row(b1), s3g, s3b, W2p, b2p, s4gp, s4bp,
                   W3p, b3p)
    return out[:, :OUT]
